# baseline (device time: 268237 ns/iter reference)
import jax
import jax.numpy as jnp
from jax import lax
from jax.experimental import pallas as pl
from jax.experimental.pallas import tpu as pltpu

N_DEV = 32


def kernel(A, B):
    m, k = A.shape
    _, n = B.shape
    m_out = m // N_DEV

    def body(a_ref, b_ref, out_ref, comm_ref, send_sems, recv_sems):
        my = lax.axis_index("i")
        left = lax.rem(my + N_DEV - 1, N_DEV)
        right = lax.rem(my + 1, N_DEV)

        barrier_sem = pltpu.get_barrier_semaphore()
        for nbr in (left, right):
            pl.semaphore_signal(
                barrier_sem, inc=1,
                device_id=(nbr,), device_id_type=pl.DeviceIdType.MESH,
            )
        pl.semaphore_wait(barrier_sem, 2)

        def partial_chunk(c):
            return jnp.dot(
                a_ref[pl.ds(c * m_out, m_out), :], b_ref[:, :],
                preferred_element_type=jnp.float32,
            )

        comm_ref[0, :, :] = partial_chunk(lax.rem(my + N_DEV - 1, N_DEV))

        for h in range(N_DEV - 1):
            rdma = pltpu.make_async_remote_copy(
                src_ref=comm_ref.at[h],
                dst_ref=comm_ref.at[h + 1],
                send_sem=send_sems.at[h],
                recv_sem=recv_sems.at[h],
                device_id=(right,),
                device_id_type=pl.DeviceIdType.MESH,
            )
            rdma.start()
            rdma.wait()
            if h < N_DEV - 2:
                c = lax.rem(my + 2 * N_DEV - 2 - h, N_DEV)
                comm_ref[h + 1, :, :] = comm_ref[h + 1, :, :] + partial_chunk(c)

        out_ref[:, :] = comm_ref[N_DEV - 1, :, :] + partial_chunk(my)

    return pl.pallas_call(
        body,
        out_shape=jax.ShapeDtypeStruct((m_out, n), jnp.float32),
        in_specs=[
            pl.BlockSpec(memory_space=pltpu.VMEM),
            pl.BlockSpec(memory_space=pltpu.VMEM),
        ],
        out_specs=pl.BlockSpec(memory_space=pltpu.VMEM),
        scratch_shapes=[
            pltpu.VMEM((N_DEV, m_out, n), jnp.float32),
            pltpu.SemaphoreType.DMA((N_DEV - 1,)),
            pltpu.SemaphoreType.DMA((N_DEV - 1,)),
        ],
        compiler_params=pltpu.CompilerParams(collective_id=0),
    )(A, B)


# device time: 245775 ns/iter; 1.0914x vs baseline; 1.0914x over previous
import jax
import jax.numpy as jnp
from jax import lax
from jax.experimental import pallas as pl
from jax.experimental.pallas import tpu as pltpu

N_DEV = 32


def kernel(A, B):
    m, k = A.shape
    _, n = B.shape
    m_out = m // N_DEV
    nh = n // 2

    def body(a_ref, b_ref, out_ref, comm1, comm2, ss1, rs1, ss2, rs2):
        my = lax.axis_index("i")
        left = lax.rem(my + N_DEV - 1, N_DEV)
        right = lax.rem(my + 1, N_DEV)

        barrier_sem = pltpu.get_barrier_semaphore()
        for nbr in (left, right):
            pl.semaphore_signal(
                barrier_sem, inc=1,
                device_id=(nbr,), device_id_type=pl.DeviceIdType.MESH,
            )
        pl.semaphore_wait(barrier_sem, 2)

        def partial_chunk(c):
            return jnp.dot(
                a_ref[pl.ds(c * m_out, m_out), :], b_ref[:, :],
                preferred_element_type=jnp.float32,
            )

        p0 = partial_chunk(lax.rem(my + N_DEV - 1, N_DEV))
        comm1[0, :, :] = p0[:, :nh]
        comm2[0, :, :] = p0[:, nh:]

        inflight = []
        for h in range(N_DEV - 1):
            r1 = pltpu.make_async_remote_copy(
                src_ref=comm1.at[h], dst_ref=comm1.at[h + 1],
                send_sem=ss1.at[h], recv_sem=rs1.at[h],
                device_id=(right,), device_id_type=pl.DeviceIdType.MESH,
            )
            r2 = pltpu.make_async_remote_copy(
                src_ref=comm2.at[h], dst_ref=comm2.at[h + 1],
                send_sem=ss2.at[h], recv_sem=rs2.at[h],
                device_id=(right,), device_id_type=pl.DeviceIdType.MESH,
            )
            r1.start()
            r2.start()
            inflight += [r1, r2]
            p = partial_chunk(lax.rem(my + 2 * N_DEV - 2 - h, N_DEV))
            r1.wait_recv()
            if h < N_DEV - 2:
                comm1[h + 1, :, :] = comm1[h + 1, :, :] + p[:, :nh]
                r2.wait_recv()
                comm2[h + 1, :, :] = comm2[h + 1, :, :] + p[:, nh:]
            else:
                out_ref[:, :nh] = comm1[N_DEV - 1, :, :] + p[:, :nh]
                r2.wait_recv()
                out_ref[:, nh:] = comm2[N_DEV - 1, :, :] + p[:, nh:]

        for r in inflight:
            r.wait_send()

    return pl.pallas_call(
        body,
        out_shape=jax.ShapeDtypeStruct((m_out, n), jnp.float32),
        in_specs=[
            pl.BlockSpec(memory_space=pltpu.VMEM),
            pl.BlockSpec(memory_space=pltpu.VMEM),
        ],
        out_specs=pl.BlockSpec(memory_space=pltpu.VMEM),
        scratch_shapes=[
            pltpu.VMEM((N_DEV, m_out, nh), jnp.float32),
            pltpu.VMEM((N_DEV, m_out, nh), jnp.float32),
            pltpu.SemaphoreType.DMA((N_DEV - 1,)),
            pltpu.SemaphoreType.DMA((N_DEV - 1,)),
            pltpu.SemaphoreType.DMA((N_DEV - 1,)),
            pltpu.SemaphoreType.DMA((N_DEV - 1,)),
        ],
        compiler_params=pltpu.CompilerParams(collective_id=0),
    )(A, B)
